# Initial kernel scaffold; baseline (speedup 1.0000x reference)
import jax
import jax.numpy as jnp
from jax.experimental import pallas as pl


def kernel(x):
    raise NotImplementedError("write your pallas kernel here")


# speedup vs baseline: 1.9340x; 1.9340x over previous
import jax
import jax.numpy as jnp
from jax.experimental import pallas as pl

_NUM_CLASSES = 1000
_BM = 512


def _onehot_concat_kernel(x_ref, o_ref):
    xb = x_ref[...]
    sel = xb[:, 0:1].astype(jnp.int32)
    cols = jax.lax.broadcasted_iota(jnp.int32, (_BM, _NUM_CLASSES), 1)
    oh = (cols == sel).astype(xb.dtype)
    o_ref[...] = jnp.concatenate([oh, xb], axis=1)


def kernel(x):
    batch, feat = x.shape
    out_w = _NUM_CLASSES + feat
    grid = (batch // _BM,)
    return pl.pallas_call(
        _onehot_concat_kernel,
        grid=grid,
        in_specs=[pl.BlockSpec((_BM, feat), lambda i: (i, 0))],
        out_specs=pl.BlockSpec((_BM, out_w), lambda i: (i, 0)),
        out_shape=jax.ShapeDtypeStruct((batch, out_w), x.dtype),
    )(x)


# speedup vs baseline: 2.1104x; 1.0912x over previous
import jax
import jax.numpy as jnp
from jax.experimental import pallas as pl

_NUM_CLASSES = 1000
_BM = 2048


def _onehot_concat_kernel(x_ref, o_ref):
    xb = x_ref[...]
    sel = xb[:, 0:1].astype(jnp.int32)
    cols = jax.lax.broadcasted_iota(jnp.int32, (_BM, _NUM_CLASSES), 1)
    oh = (cols == sel).astype(xb.dtype)
    o_ref[...] = jnp.concatenate([oh, xb], axis=1)


def kernel(x):
    batch, feat = x.shape
    out_w = _NUM_CLASSES + feat
    grid = (batch // _BM,)
    return pl.pallas_call(
        _onehot_concat_kernel,
        grid=grid,
        in_specs=[pl.BlockSpec((_BM, feat), lambda i: (i, 0))],
        out_specs=pl.BlockSpec((_BM, out_w), lambda i: (i, 0)),
        out_shape=jax.ShapeDtypeStruct((batch, out_w), x.dtype),
    )(x)
